# baseline (speedup 1.0000x reference)
import functools

import jax
import jax.numpy as jnp
from jax import lax
from jax.experimental import pallas as pl
from jax.experimental.pallas import tpu as pltpu
from jax.experimental.pallas import tpu_sc as plsc

IN_CH = 256
HID = 256
NCLS = 10
NTOT = 224 * 224
NCORE = 2
NSUB = 16
NW = NCORE * NSUB
CHUNK = NTOT // NW
NVREG = CHUNK // 16
CAP_CORE = NSUB * CHUNK
NC_ROWS = NCORE * CAP_CORE
NFULL = NTOT + 8
BLKC = 512
BPC = CAP_CORE // BLKC
NBLK = NC_ROWS // BLKC
IDX_BUF = CHUNK + 32

_MESH = plsc.VectorSubcoreMesh(core_axis_name="c", subcore_axis_name="s",
                               num_cores=NCORE, num_subcores=NSUB)
_IOTA16 = lambda: lax.iota(jnp.int32, 16)


def _lane0(v):
    return jnp.sum(jnp.where(_IOTA16() == 0, v, 0))


def _compact_body(mask_hbm, x_hbm,
                  xc_hbm, idxs_hbm, counts_hbm, offs_hbm, cnts_hbm,
                  mv_ref, idxg_ref, idxs_ref, row_buf, row_buf2, v16_ref,
                  sh_loc, shared, sem, sem2):
    c = lax.axis_index("c")
    s = lax.axis_index("s")
    w = c * NSUB + s
    base = w * CHUNK

    def _init(i, _):
        idxg_ref[pl.ds(i * 16, 16)] = jnp.zeros((16,), jnp.int32)
        idxs_ref[pl.ds(i * 16, 16)] = jnp.full((16,), NTOT, jnp.int32)
        return 0
    lax.fori_loop(0, IDX_BUF // 16, _init, 0)

    pltpu.sync_copy(mask_hbm.at[pl.ds(pl.multiple_of(base, 8), CHUNK)],
                    mv_ref)

    def _step(i, off):
        mv = mv_ref[pl.ds(i * 16, 16)]
        m = mv != 0
        mi = jnp.where(m, 1, 0)
        csum = plsc.cumsum(mi)
        gidx = base + i * 16 + _IOTA16()
        pos = jnp.where(m, off + csum - 1, IDX_BUF - 8)
        plsc.store_scatter(idxg_ref, [pos], gidx)
        plsc.store_scatter(idxs_ref, [pos], gidx)
        return off + jnp.sum(mi)
    cnt = lax.fori_loop(0, NVREG, _step, 0)
    cnt16 = ((cnt + 15) // 16) * 16

    v16_ref[...] = jnp.full((16,), cnt16, jnp.int32)
    pltpu.sync_copy(v16_ref, shared.at[pl.ds(s * 16, 16)])
    plsc.subcore_barrier()
    pltpu.sync_copy(shared, sh_loc)
    ci = plsc.load_gather(sh_loc, [_IOTA16() * 16])
    off_core = jnp.sum(jnp.where(_IOTA16() < s, ci, 0))
    ctot = jnp.sum(ci)
    abs_off = c * CAP_CORE + off_core

    v16_ref[...] = jnp.full((16,), abs_off, jnp.int32)
    pltpu.sync_copy(v16_ref, offs_hbm.at[pl.ds(pl.multiple_of(w * 16, 16),
                                               16)])
    v16_ref[...] = jnp.full((16,), cnt, jnp.int32)
    pltpu.sync_copy(v16_ref, cnts_hbm.at[pl.ds(pl.multiple_of(w * 16, 16),
                                               16)])

    @pl.when(s == 0)
    def _():
        v16_ref[...] = jnp.full((16,), ctot, jnp.int32)
        pltpu.sync_copy(v16_ref,
                        counts_hbm.at[pl.ds(pl.multiple_of(c * 16, 16), 16)])

    def _gather(src_off, buf, n, gsem):
        return pltpu.make_async_copy(
            x_hbm.at[idxg_ref.at[pl.ds(src_off, n)]],
            buf.at[pl.ds(0, n)], gsem)

    def _wback(dst_off, buf, n):
        pltpu.sync_copy(buf.at[pl.ds(0, n)],
                        xc_hbm.at[pl.ds(pl.multiple_of(dst_off, 16), n)])

    bufs = (row_buf, row_buf2)
    sems = (sem, sem2)
    NP = CHUNK // 128

    @pl.when(128 <= cnt16)
    def _():
        _gather(0, bufs[0], 128, sems[0]).start()
    for k in range(NP):
        if k + 1 < NP:
            @pl.when((k + 2) * 128 <= cnt16)
            def _():
                _gather((k + 1) * 128, bufs[(k + 1) % 2],
                        128, sems[(k + 1) % 2]).start()

        @pl.when((k + 1) * 128 <= cnt16)
        def _():
            pltpu.sync_copy(idxs_ref.at[pl.ds(k * 128, 128)],
                            idxs_hbm.at[pl.ds(pl.multiple_of(
                                abs_off + k * 128, 16), 128)])
            _gather(k * 128, bufs[k % 2], 128, sems[k % 2]).wait()
            _wback(abs_off + k * 128, bufs[k % 2], 128)
    t0 = (cnt16 // 128) * 128
    for k in range(8):
        @pl.when(t0 + k * 16 < cnt16)
        def _():
            pltpu.sync_copy(idxs_ref.at[pl.ds(t0 + k * 16, 16)],
                            idxs_hbm.at[pl.ds(pl.multiple_of(
                                abs_off + t0 + k * 16, 16), 16)])
            g = _gather(t0 + k * 16, bufs[k % 2], 16, sems[k % 2])
            g.start()
            g.wait()
            _wback(abs_off + t0 + k * 16, bufs[k % 2], 16)


def _sc_compact_gather(mask_i32, x2):
    f = functools.partial(
        pl.kernel,
        out_type=(
            jax.ShapeDtypeStruct((NC_ROWS, IN_CH), jnp.float32),
            jax.ShapeDtypeStruct((NC_ROWS,), jnp.int32),
            jax.ShapeDtypeStruct((NCORE * 16,), jnp.int32),
            jax.ShapeDtypeStruct((NW * 16,), jnp.int32),
            jax.ShapeDtypeStruct((NW * 16,), jnp.int32),
        ),
        mesh=_MESH,
        compiler_params=pltpu.CompilerParams(needs_layout_passes=False),
        scratch_types=[
            pltpu.VMEM((CHUNK,), jnp.int32),
            pltpu.VMEM((IDX_BUF,), jnp.int32),
            pltpu.VMEM((IDX_BUF,), jnp.int32),
            pltpu.VMEM((128, IN_CH), jnp.float32),
            pltpu.VMEM((128, IN_CH), jnp.float32),
            pltpu.VMEM((16,), jnp.int32),
            pltpu.VMEM((NSUB * 16,), jnp.int32),
            pltpu.VMEM_SHARED((NSUB * 16,), jnp.int32),
            pltpu.SemaphoreType.DMA,
            pltpu.SemaphoreType.DMA,
        ],
    )
    return f(_compact_body)(mask_i32, x2)


def _ln(h, g, b):
    m = jnp.mean(h, axis=-1, keepdims=True)
    v = jnp.mean((h - m) ** 2, axis=-1, keepdims=True)
    return (h - m) * lax.rsqrt(v + 1e-5) * g + b


def _head(xb, W0, g0, b0, W1, g1, b1, Wout):
    h = jnp.dot(xb, W0, preferred_element_type=jnp.float32)
    h = jnp.maximum(_ln(h, g0, b0), 0.0).astype(jnp.bfloat16)
    h = jnp.dot(h, W1, preferred_element_type=jnp.float32)
    h = jnp.maximum(_ln(h, g1, b1), 0.0).astype(jnp.bfloat16)
    return jnp.dot(h, Wout, preferred_element_type=jnp.float32)


def _mlp_body(cnt_ref, xc_ref,
              cW0, cg0, cb0, cW1, cg1, cb1, cWo,
              rW0, rg0, rb0, rW1, rg1, rb1, rWo,
              dW0, dg0, db0, dW1, dg1, db1, dWo,
              bsml,
              ys_ref):
    j = pl.program_id(0)
    r = j // BPC
    p = j % BPC

    @pl.when(p * BLKC < cnt_ref[r * 16])
    def _():
        xb = xc_ref[...].astype(jnp.bfloat16)
        o_c = _head(xb, cW0[...], cg0[...], cb0[...], cW1[...], cg1[...],
                    cb1[...], cWo[...])
        o_r = _head(xb, rW0[...], rg0[...], rb0[...], rW1[...], rg1[...],
                    rb1[...], rWo[...])
        o_d = _head(xb, dW0[...], dg0[...], db0[...], dW1[...], dg1[...],
                    db1[...], dWo[...])
        zeros2 = jnp.zeros((BLKC, 2), jnp.float32)
        ys_ref[...] = jnp.concatenate([o_c, o_r, o_d, zeros2],
                                      axis=1) + bsml[...]


def _tc_mlp(counts, xc, params):
    def head_args(p):
        return [p['W0'].astype(jnp.bfloat16),
                jnp.reshape(p['g0'], (1, HID)), jnp.reshape(p['b0'], (1, HID)),
                p['W1'].astype(jnp.bfloat16),
                jnp.reshape(p['g1'], (1, HID)), jnp.reshape(p['b1'], (1, HID)),
                p['Wout'].astype(jnp.bfloat16)]

    bsml = jnp.concatenate([params['cls']['bout'], params['reg']['bout'],
                            params['dir']['bout'],
                            jnp.zeros((2,), jnp.float32)])
    bsml = jnp.reshape(bsml, (1, 16))
    weights = (head_args(params['cls']) + head_args(params['reg'])
               + head_args(params['dir'])
               + [bsml])

    def wspec(w):
        return pl.BlockSpec(w.shape, lambda j, cnt: (0, 0))

    def xmap(j, cnt):
        r = j // BPC
        p = j % BPC
        c = cnt[r * 16]
        cap = jnp.maximum((c + BLKC - 1) // BLKC - 1, 0)
        return (r * BPC + jnp.minimum(p, cap), 0)

    def omap(j, cnt):
        r = j // BPC
        p = j % BPC
        c = cnt[r * 16]
        return (jnp.where(p * BLKC < c, j, NBLK), 0)

    grid_spec = pltpu.PrefetchScalarGridSpec(
        num_scalar_prefetch=1,
        grid=(NC_ROWS // BLKC,),
        in_specs=[pl.BlockSpec((BLKC, IN_CH), xmap)]
                 + [wspec(w) for w in weights],
        out_specs=(pl.BlockSpec((BLKC, 16), omap),),
    )
    return pl.pallas_call(
        _mlp_body,
        grid_spec=grid_spec,
        out_shape=(jax.ShapeDtypeStruct((NC_ROWS + BLKC, 16), jnp.float32),),
        compiler_params=pltpu.CompilerParams(
            dimension_semantics=("arbitrary",)),
    )(counts, xc, *weights)


def _feat_body(x_ref, mask_ref, W0T, g0, b0, W1T, g1, b1, WoT, out_ref):
    xb = x_ref[...].astype(jnp.bfloat16)
    m = mask_ref[...]
    h = lax.dot_general(W0T[...], xb, (((1,), (1,)), ((), ())),
                        preferred_element_type=jnp.float32)
    h = jnp.maximum(_ln_t(h, g0[...], b0[...]), 0.0).astype(jnp.bfloat16)
    h = jnp.dot(W1T[...], h, preferred_element_type=jnp.float32)
    h = jnp.maximum(_ln_t(h, g1[...], b1[...]), 0.0).astype(jnp.bfloat16)
    o = jnp.dot(WoT[...], h, preferred_element_type=jnp.float32)
    out_ref[...] = o * m


def _ln_t(h, g, b):
    m = jnp.mean(h, axis=0, keepdims=True)
    v = jnp.mean((h - m) ** 2, axis=0, keepdims=True)
    return (h - m) * lax.rsqrt(v + 1e-5) * g + b


BLKF = 512


def _tc_feat(x2, mask_f, params):
    p = params['feat']
    w = [p['W0'].T.astype(jnp.bfloat16),
         jnp.reshape(p['g0'], (HID, 1)), jnp.reshape(p['b0'], (HID, 1)),
         p['W1'].T.astype(jnp.bfloat16),
         jnp.reshape(p['g1'], (HID, 1)), jnp.reshape(p['b1'], (HID, 1)),
         p['Wout'].T.astype(jnp.bfloat16)]

    def wspec(a):
        return pl.BlockSpec(a.shape, lambda j: (0, 0))

    return pl.pallas_call(
        _feat_body,
        grid=(NTOT // BLKF,),
        in_specs=[pl.BlockSpec((BLKF, IN_CH), lambda j: (j, 0)),
                  pl.BlockSpec((1, BLKF), lambda j: (0, j))]
                 + [wspec(a) for a in w],
        out_specs=pl.BlockSpec((HID, BLKF), lambda j: (0, j)),
        out_shape=jax.ShapeDtypeStruct((HID, NTOT), jnp.float32),
        compiler_params=pltpu.CompilerParams(
            dimension_semantics=("arbitrary",)),
    )(x2, mask_f, *w)


def _scatter_body(ysc_hbm, idxs_hbm, offs_hbm, cnts_hbm,
                  ysf_hbm,
                  v16_ref, idxa_ref, smc_ref, sms_ref, sem):
    c = lax.axis_index("c")
    s = lax.axis_index("s")
    w = c * NSUB + s
    base = w * CHUNK

    pltpu.sync_copy(offs_hbm.at[pl.ds(pl.multiple_of(w * 16, 16), 16)],
                    v16_ref)
    off_w = pl.multiple_of(_lane0(v16_ref[...]), 16)
    pltpu.sync_copy(cnts_hbm.at[pl.ds(pl.multiple_of(w * 16, 16), 16)],
                    v16_ref)
    cnt = _lane0(v16_ref[...])

    pltpu.sync_copy(idxs_hbm.at[pl.ds(off_w, CHUNK)], idxa_ref)
    pltpu.sync_copy(ysc_hbm.at[pl.ds(pl.multiple_of(off_w * 16, 128),
                                     CHUNK * 16)], smc_ref)

    def _zsml(i, _):
        sms_ref[pl.ds(i * 16, 16)] = jnp.zeros((16,), jnp.float32)
        return 0
    lax.fori_loop(0, CHUNK, _zsml, 0)

    def _place(g, _):
        iv = idxa_ref[pl.ds(g * 16, 16)]
        for l in range(16):
            @pl.when(g * 16 + l < cnt)
            def _():
                li = iv[l] - base
                sms_ref[pl.ds(li * 16, 16)] = smc_ref[pl.ds((g * 16 + l) * 16,
                                                            16)]
        return 0
    lax.fori_loop(0, (cnt + 15) // 16, _place, 0)
    pltpu.sync_copy(sms_ref,
                    ysf_hbm.at[pl.ds(pl.multiple_of(base * 16, 128),
                                     CHUNK * 16)])


def _sc_scatter(ysc, idxs, offs, cnts):
    f = functools.partial(
        pl.kernel,
        out_type=jax.ShapeDtypeStruct((NTOT * 16,), jnp.float32),
        mesh=_MESH,
        compiler_params=pltpu.CompilerParams(needs_layout_passes=False),
        scratch_types=[
            pltpu.VMEM((16,), jnp.int32),
            pltpu.VMEM((CHUNK,), jnp.int32),
            pltpu.VMEM((CHUNK * 16,), jnp.float32),
            pltpu.VMEM((CHUNK * 16,), jnp.float32),
            pltpu.SemaphoreType.DMA,
        ],
    )
    return f(_scatter_body)(jnp.reshape(ysc, ((NC_ROWS + BLKC) * 16,)),
                            idxs, offs, cnts)


def kernel(x, active_points, params):
    B, H, W, _ = x.shape
    x2 = jnp.reshape(x, (NTOT, IN_CH))
    mask_i32 = jnp.reshape(active_points, (NTOT,)).astype(jnp.int32)
    mask_f = jnp.reshape(active_points, (1, NTOT)).astype(jnp.float32)

    xc, idxs, counts, offs, cnts = _sc_compact_gather(mask_i32, x2)
    (ysc,) = _tc_mlp(counts, xc, params)
    featT = _tc_feat(x2, mask_f, params)
    ysf = _sc_scatter(ysc, idxs, offs, cnts)

    feat = jnp.reshape(featT, (B, HID, H, W))
    small = jnp.transpose(jnp.reshape(ysf, (B, H, W, 16)), (0, 3, 1, 2))
    return (small[:, :NCLS], small[:, NCLS:NCLS + 2],
            small[:, NCLS + 2:NCLS + 4], feat)


# speedup vs baseline: 1.1531x; 1.1531x over previous
import functools

import jax
import jax.numpy as jnp
from jax import lax
from jax.experimental import pallas as pl
from jax.experimental.pallas import tpu as pltpu
from jax.experimental.pallas import tpu_sc as plsc

IN_CH = 256
HID = 256
NCLS = 10
NTOT = 224 * 224
NCORE = 2
NSUB = 16
NW = NCORE * NSUB
CHUNK = NTOT // NW
NVREG = CHUNK // 16
CAP_CORE = NSUB * CHUNK
NC_ROWS = NCORE * CAP_CORE
NFULL = NTOT + 8
BLKC = 512
BPC = CAP_CORE // BLKC
NBLK = NC_ROWS // BLKC
IDX_BUF = CHUNK + 32

_MESH = plsc.VectorSubcoreMesh(core_axis_name="c", subcore_axis_name="s",
                               num_cores=NCORE, num_subcores=NSUB)
_IOTA16 = lambda: lax.iota(jnp.int32, 16)


def _lane0(v):
    return jnp.sum(jnp.where(_IOTA16() == 0, v, 0))


def _compact_body(mask_hbm, x_hbm,
                  xc_hbm, idxs_hbm, counts_hbm, offs_hbm, cnts_hbm,
                  mv_ref, idxg_ref, idxs_ref, row_buf, row_buf2, v16_ref,
                  sh_loc, shared, sem, sem2):
    c = lax.axis_index("c")
    s = lax.axis_index("s")
    w = c * NSUB + s
    base = w * CHUNK

    def _init(i, _):
        idxg_ref[pl.ds(i * 16, 16)] = jnp.zeros((16,), jnp.int32)
        idxs_ref[pl.ds(i * 16, 16)] = jnp.full((16,), NTOT, jnp.int32)
        return 0
    lax.fori_loop(0, IDX_BUF // 16, _init, 0)

    pltpu.sync_copy(mask_hbm.at[pl.ds(pl.multiple_of(base, 8), CHUNK)],
                    mv_ref)

    def _step(i, off):
        mv = mv_ref[pl.ds(i * 16, 16)]
        m = mv != 0
        mi = jnp.where(m, 1, 0)
        csum = plsc.cumsum(mi)
        gidx = base + i * 16 + _IOTA16()
        pos = jnp.where(m, off + csum - 1, IDX_BUF - 8)
        plsc.store_scatter(idxg_ref, [pos], gidx)
        plsc.store_scatter(idxs_ref, [pos], gidx)
        return off + jnp.sum(mi)
    cnt = lax.fori_loop(0, NVREG, _step, 0)
    cnt16 = ((cnt + 15) // 16) * 16

    v16_ref[...] = jnp.full((16,), cnt16, jnp.int32)
    pltpu.sync_copy(v16_ref, shared.at[pl.ds(s * 16, 16)])
    plsc.subcore_barrier()
    pltpu.sync_copy(shared, sh_loc)
    ci = plsc.load_gather(sh_loc, [_IOTA16() * 16])
    off_core = jnp.sum(jnp.where(_IOTA16() < s, ci, 0))
    ctot = jnp.sum(ci)
    abs_off = c * CAP_CORE + off_core

    v16_ref[...] = jnp.full((16,), abs_off, jnp.int32)
    pltpu.sync_copy(v16_ref, offs_hbm.at[pl.ds(pl.multiple_of(w * 16, 16),
                                               16)])
    v16_ref[...] = jnp.full((16,), cnt, jnp.int32)
    pltpu.sync_copy(v16_ref, cnts_hbm.at[pl.ds(pl.multiple_of(w * 16, 16),
                                               16)])

    @pl.when(s == 0)
    def _():
        v16_ref[...] = jnp.full((16,), ctot, jnp.int32)
        pltpu.sync_copy(v16_ref,
                        counts_hbm.at[pl.ds(pl.multiple_of(c * 16, 16), 16)])

    def _gather(src_off, buf, n, gsem):
        return pltpu.make_async_copy(
            x_hbm.at[idxg_ref.at[pl.ds(src_off, n)]],
            buf.at[pl.ds(0, n)], gsem)

    def _wback(dst_off, buf, n):
        pltpu.sync_copy(buf.at[pl.ds(0, n)],
                        xc_hbm.at[pl.ds(pl.multiple_of(dst_off, 16), n)])

    bufs = (row_buf, row_buf2)
    sems = (sem, sem2)
    NP = CHUNK // 128

    @pl.when(128 <= cnt16)
    def _():
        _gather(0, bufs[0], 128, sems[0]).start()
    for k in range(NP):
        if k + 1 < NP:
            @pl.when((k + 2) * 128 <= cnt16)
            def _():
                _gather((k + 1) * 128, bufs[(k + 1) % 2],
                        128, sems[(k + 1) % 2]).start()

        @pl.when((k + 1) * 128 <= cnt16)
        def _():
            pltpu.sync_copy(idxs_ref.at[pl.ds(k * 128, 128)],
                            idxs_hbm.at[pl.ds(pl.multiple_of(
                                abs_off + k * 128, 16), 128)])
            _gather(k * 128, bufs[k % 2], 128, sems[k % 2]).wait()
            _wback(abs_off + k * 128, bufs[k % 2], 128)
    t0 = (cnt16 // 128) * 128
    for k in range(8):
        @pl.when(t0 + k * 16 < cnt16)
        def _():
            pltpu.sync_copy(idxs_ref.at[pl.ds(t0 + k * 16, 16)],
                            idxs_hbm.at[pl.ds(pl.multiple_of(
                                abs_off + t0 + k * 16, 16), 16)])
            g = _gather(t0 + k * 16, bufs[k % 2], 16, sems[k % 2])
            g.start()
            g.wait()
            _wback(abs_off + t0 + k * 16, bufs[k % 2], 16)


def _sc_compact_gather(mask_i32, x2):
    f = functools.partial(
        pl.kernel,
        out_type=(
            jax.ShapeDtypeStruct((NC_ROWS, IN_CH), jnp.float32),
            jax.ShapeDtypeStruct((NC_ROWS,), jnp.int32),
            jax.ShapeDtypeStruct((NCORE * 16,), jnp.int32),
            jax.ShapeDtypeStruct((NW * 16,), jnp.int32),
            jax.ShapeDtypeStruct((NW * 16,), jnp.int32),
        ),
        mesh=_MESH,
        compiler_params=pltpu.CompilerParams(needs_layout_passes=False),
        scratch_types=[
            pltpu.VMEM((CHUNK,), jnp.int32),
            pltpu.VMEM((IDX_BUF,), jnp.int32),
            pltpu.VMEM((IDX_BUF,), jnp.int32),
            pltpu.VMEM((128, IN_CH), jnp.float32),
            pltpu.VMEM((128, IN_CH), jnp.float32),
            pltpu.VMEM((16,), jnp.int32),
            pltpu.VMEM((NSUB * 16,), jnp.int32),
            pltpu.VMEM_SHARED((NSUB * 16,), jnp.int32),
            pltpu.SemaphoreType.DMA,
            pltpu.SemaphoreType.DMA,
        ],
    )
    return f(_compact_body)(mask_i32, x2)


def _ln(h, g, b):
    m = jnp.mean(h, axis=-1, keepdims=True)
    v = jnp.mean((h - m) ** 2, axis=-1, keepdims=True)
    return (h - m) * lax.rsqrt(v + 1e-5) * g + b


def _head(xb, W0, g0, b0, W1, g1, b1, Wout):
    h = jnp.dot(xb, W0, preferred_element_type=jnp.float32)
    h = jnp.maximum(_ln(h, g0, b0), 0.0).astype(jnp.bfloat16)
    h = jnp.dot(h, W1, preferred_element_type=jnp.float32)
    h = jnp.maximum(_ln(h, g1, b1), 0.0).astype(jnp.bfloat16)
    return jnp.dot(h, Wout, preferred_element_type=jnp.float32)


def _mlp_body(cnt_ref, xc_ref,
              cW0, cg0, cb0, cW1, cg1, cb1, cWo,
              rW0, rg0, rb0, rW1, rg1, rb1, rWo,
              dW0, dg0, db0, dW1, dg1, db1, dWo,
              fW0, fg0, fb0, fW1, fg1, fb1, fWo,
              bsml,
              yf_ref, ys_ref):
    j = pl.program_id(0)
    r = j // BPC
    p = j % BPC

    @pl.when(p * BLKC < cnt_ref[r * 16])
    def _():
        xb = xc_ref[...].astype(jnp.bfloat16)
        o_c = _head(xb, cW0[...], cg0[...], cb0[...], cW1[...], cg1[...],
                    cb1[...], cWo[...])
        o_r = _head(xb, rW0[...], rg0[...], rb0[...], rW1[...], rg1[...],
                    rb1[...], rWo[...])
        o_d = _head(xb, dW0[...], dg0[...], db0[...], dW1[...], dg1[...],
                    db1[...], dWo[...])
        o_f = _head(xb, fW0[...], fg0[...], fb0[...], fW1[...], fg1[...],
                    fb1[...], fWo[...])
        zeros2 = jnp.zeros((BLKC, 2), jnp.float32)
        ys_ref[...] = jnp.concatenate([o_c, o_r, o_d, zeros2],
                                      axis=1) + bsml[...]
        yf_ref[...] = o_f


def _tc_mlp(counts, xc, params):
    def head_args(p):
        return [p['W0'].astype(jnp.bfloat16),
                jnp.reshape(p['g0'], (1, HID)), jnp.reshape(p['b0'], (1, HID)),
                p['W1'].astype(jnp.bfloat16),
                jnp.reshape(p['g1'], (1, HID)), jnp.reshape(p['b1'], (1, HID)),
                p['Wout'].astype(jnp.bfloat16)]

    bsml = jnp.concatenate([params['cls']['bout'], params['reg']['bout'],
                            params['dir']['bout'],
                            jnp.zeros((2,), jnp.float32)])
    bsml = jnp.reshape(bsml, (1, 16))
    weights = (head_args(params['cls']) + head_args(params['reg'])
               + head_args(params['dir']) + head_args(params['feat'])
               + [bsml])

    def wspec(w):
        return pl.BlockSpec(w.shape, lambda j, cnt: (0, 0))

    def xmap(j, cnt):
        r = j // BPC
        p = j % BPC
        c = cnt[r * 16]
        cap = jnp.maximum((c + BLKC - 1) // BLKC - 1, 0)
        return (r * BPC + jnp.minimum(p, cap), 0)

    def omap(j, cnt):
        r = j // BPC
        p = j % BPC
        c = cnt[r * 16]
        return (jnp.where(p * BLKC < c, j, NBLK), 0)

    grid_spec = pltpu.PrefetchScalarGridSpec(
        num_scalar_prefetch=1,
        grid=(NC_ROWS // BLKC,),
        in_specs=[pl.BlockSpec((BLKC, IN_CH), xmap)]
                 + [wspec(w) for w in weights],
        out_specs=(pl.BlockSpec((BLKC, HID), omap),
                   pl.BlockSpec((BLKC, 16), omap)),
    )
    return pl.pallas_call(
        _mlp_body,
        grid_spec=grid_spec,
        out_shape=(jax.ShapeDtypeStruct((NC_ROWS + BLKC, HID), jnp.float32),
                   jax.ShapeDtypeStruct((NC_ROWS + BLKC, 16), jnp.float32)),
        compiler_params=pltpu.CompilerParams(
            dimension_semantics=("arbitrary",)),
    )(counts, xc, *weights)


def _scatter_body(yfc_hbm, ysc_hbm, idxs_hbm, offs_hbm, cnts_hbm,
                  yff_hbm, ysf_hbm,
                  dbf_ref, dbf2_ref, idxv_ref, idxv2_ref,
                  idxv16_ref, v16_ref,
                  idxa_ref, smc_ref, sms_ref, sem, lsem0, lsem1):
    c = lax.axis_index("c")
    s = lax.axis_index("s")
    w = c * NSUB + s
    base = w * CHUNK

    pltpu.sync_copy(offs_hbm.at[pl.ds(pl.multiple_of(w * 16, 16), 16)],
                    v16_ref)
    off_w = pl.multiple_of(_lane0(v16_ref[...]), 16)
    pltpu.sync_copy(cnts_hbm.at[pl.ds(pl.multiple_of(w * 16, 16), 16)],
                    v16_ref)
    cnt = _lane0(v16_ref[...])
    cnt16 = ((cnt + 15) // 16) * 16

    pltpu.sync_copy(idxs_hbm.at[pl.ds(off_w, CHUNK)], idxa_ref)
    pltpu.sync_copy(ysc_hbm.at[pl.ds(pl.multiple_of(off_w * 16, 128),
                                     CHUNK * 16)], smc_ref)

    def _zsml(i, _):
        sms_ref[pl.ds(i * 16, 16)] = jnp.zeros((16,), jnp.float32)
        return 0
    lax.fori_loop(0, CHUNK, _zsml, 0)

    def _place(g, _):
        iv = idxa_ref[pl.ds(g * 16, 16)]
        for l in range(16):
            @pl.when(g * 16 + l < cnt)
            def _():
                li = iv[l] - base
                sms_ref[pl.ds(li * 16, 16)] = smc_ref[pl.ds((g * 16 + l) * 16,
                                                            16)]
        return 0
    lax.fori_loop(0, (cnt + 15) // 16, _place, 0)
    pltpu.sync_copy(sms_ref,
                    ysf_hbm.at[pl.ds(pl.multiple_of(base * 16, 128),
                                     CHUNK * 16)])

    def _zrow(i, _):
        def _zcol(l, __):
            dbf_ref[i, pl.ds(l * 16, 16)] = jnp.zeros((16,), jnp.float32)
            return 0
        lax.fori_loop(0, IN_CH // 16, _zcol, 0)
        return 0
    lax.fori_loop(0, 128, _zrow, 0)
    for k in range(CHUNK // 128):
        pltpu.sync_copy(dbf_ref, yff_hbm.at[pl.ds(base + k * 128, 128)])
    pltpu.sync_copy(dbf_ref.at[pl.ds(0, 32)],
                    yff_hbm.at[pl.ds(base + 1536, 32)])

    def _load(src_off, idx_ref, buf, n, lsem):
        src_off = pl.multiple_of(src_off, 16)
        pltpu.sync_copy(idxs_hbm.at[pl.ds(src_off, n)], idx_ref)
        pltpu.make_async_copy(yfc_hbm.at[pl.ds(src_off, n)],
                              buf.at[pl.ds(0, n)], lsem).start()

    def _scat(idx_ref, buf, n, lsem):
        pltpu.make_async_copy(yfc_hbm.at[pl.ds(0, n)],
                              buf.at[pl.ds(0, n)], lsem).wait()
        pltpu.async_copy(buf.at[pl.ds(0, n)], yff_hbm.at[idx_ref],
                         sem).wait()

    NP = CHUNK // 128
    bufs = (dbf_ref, dbf2_ref)
    idxr = (idxv_ref, idxv2_ref)
    lsems = (lsem0, lsem1)

    @pl.when(128 <= cnt16)
    def _():
        _load(off_w, idxr[0], bufs[0], 128, lsems[0])
    for k in range(NP):
        if k + 1 < NP:
            @pl.when((k + 2) * 128 <= cnt16)
            def _():
                _load(off_w + (k + 1) * 128, idxr[(k + 1) % 2],
                      bufs[(k + 1) % 2], 128, lsems[(k + 1) % 2])

        @pl.when((k + 1) * 128 <= cnt16)
        def _():
            _scat(idxr[k % 2], bufs[k % 2], 128, lsems[k % 2])
    t0 = (cnt16 // 128) * 128
    for k in range(8):
        @pl.when(t0 + k * 16 < cnt16)
        def _():
            _load(off_w + t0 + k * 16, idxv16_ref, bufs[k % 2], 16,
                  lsems[k % 2])
            _scat(idxv16_ref, bufs[k % 2], 16, lsems[k % 2])


def _sc_scatter(yfc, ysc, idxs, offs, cnts):
    f = functools.partial(
        pl.kernel,
        out_type=(
            jax.ShapeDtypeStruct((NFULL, HID), jnp.float32),
            jax.ShapeDtypeStruct((NFULL * 16,), jnp.float32),
        ),
        mesh=_MESH,
        compiler_params=pltpu.CompilerParams(needs_layout_passes=False),
        scratch_types=[
            pltpu.VMEM((128, HID), jnp.float32),
            pltpu.VMEM((128, HID), jnp.float32),
            pltpu.VMEM((128,), jnp.int32),
            pltpu.VMEM((128,), jnp.int32),
            pltpu.VMEM((16,), jnp.int32),
            pltpu.VMEM((16,), jnp.int32),
            pltpu.VMEM((CHUNK,), jnp.int32),
            pltpu.VMEM((CHUNK * 16,), jnp.float32),
            pltpu.VMEM((CHUNK * 16,), jnp.float32),
            pltpu.SemaphoreType.DMA,
            pltpu.SemaphoreType.DMA,
            pltpu.SemaphoreType.DMA,
        ],
    )
    return f(_scatter_body)(yfc, jnp.reshape(ysc, ((NC_ROWS + BLKC) * 16,)),
                            idxs, offs, cnts)


def kernel(x, active_points, params):
    B, H, W, _ = x.shape
    x2 = jnp.reshape(x, (NTOT, IN_CH))
    mask_i32 = jnp.reshape(active_points, (NTOT,)).astype(jnp.int32)

    xc, idxs, counts, offs, cnts = _sc_compact_gather(mask_i32, x2)
    yfc, ysc = _tc_mlp(counts, xc, params)
    yff, ysf = _sc_scatter(yfc, ysc, idxs, offs, cnts)

    feat = jnp.transpose(jnp.reshape(yff[:NTOT], (B, H, W, HID)), (0, 3, 1, 2))
    small = jnp.transpose(
        jnp.reshape(ysf[:NTOT * 16], (B, H, W, 16)), (0, 3, 1, 2))
    return (small[:, :NCLS], small[:, NCLS:NCLS + 2],
            small[:, NCLS + 2:NCLS + 4], feat)
